# baseline (speedup 1.0000x reference)
import functools

import numpy as np
import jax
from jax import lax
import jax.numpy as jnp
from jax.experimental import pallas as pl
from jax.experimental.pallas import tpu as pltpu
from jax.experimental.pallas import tpu_sc as plsc

N = 4096
RAD_ORDER = 16
RAD_CUTOFF = 6.0
ANG_ORDER = 8
ANG_CUTOFF = 4.0
MIN_CUTOFF = 0.55
M_ANG = 48
BIG = 1e30
VTHRESH = 1e20

BI = 256
BJ = 512
BA = 64
MSLOT = 64

NC = 2
NS = 16
NW = NC * NS
ROWS_PER = N // NW
LANE = 16
RBATCH = 4


def _dang_kernel(pos_ref, posT_ref, dang_ref):
    i_blk = pl.program_id(0)
    j_blk = pl.program_id(1)
    xi = pos_ref[:, 0:1]
    yi = pos_ref[:, 1:2]
    zi = pos_ref[:, 2:3]
    xj = posT_ref[0:1, :]
    yj = posT_ref[1:2, :]
    zj = posT_ref[2:3, :]
    dx = xi - xj
    dy = yi - yj
    dz = zi - zj
    d = jnp.sqrt(dx * dx + dy * dy + dz * dz + 1e-12)
    gi = i_blk * BI + jax.lax.broadcasted_iota(jnp.int32, (BI, BJ), 0)
    gj = j_blk * BJ + jax.lax.broadcasted_iota(jnp.int32, (BI, BJ), 1)
    notdiag = gi != gj
    mang = (d <= ANG_CUTOFF) & (d > MIN_CUTOFF) & notdiag
    dang_ref[...] = jnp.where(mang, d, BIG)


def _rad_kernel(pos_ref, posT_ref, spinT_ref, radu_ref, radw_ref):
    i_blk = pl.program_id(0)
    j_blk = pl.program_id(1)
    xi = pos_ref[:, 0:1]
    yi = pos_ref[:, 1:2]
    zi = pos_ref[:, 2:3]
    xj = posT_ref[0:1, :]
    yj = posT_ref[1:2, :]
    zj = posT_ref[2:3, :]
    dx = xi - xj
    dy = yi - yj
    dz = zi - zj
    d = jnp.sqrt(dx * dx + dy * dy + dz * dz + 1e-12)
    gi = i_blk * BI + jax.lax.broadcasted_iota(jnp.int32, (BI, BJ), 0)
    gj = j_blk * BJ + jax.lax.broadcasted_iota(jnp.int32, (BI, BJ), 1)
    notdiag = gi != gj
    mrad = (d <= RAD_CUTOFF) & (d > MIN_CUTOFF) & notdiag
    w = jnp.where(mrad, 0.5 * (jnp.cos(jnp.float32(np.pi / RAD_CUTOFF) * d) + 1.0), 0.0)
    ws = w * spinT_ref[0:1, :]
    x = jnp.where(mrad, 2.0 * (d - MIN_CUTOFF) / (RAD_CUTOFF - MIN_CUTOFF) - 1.0, 0.0)
    su = [jnp.sum(w, axis=1, keepdims=True), jnp.sum(w * x, axis=1, keepdims=True)]
    sw = [jnp.sum(ws, axis=1, keepdims=True), jnp.sum(ws * x, axis=1, keepdims=True)]
    tkm1 = jnp.ones_like(x)
    tk = x
    for _ in range(2, RAD_ORDER + 1):
        tnew = 2.0 * x * tk - tkm1
        tkm1, tk = tk, tnew
        su.append(jnp.sum(w * tnew, axis=1, keepdims=True))
        sw.append(jnp.sum(ws * tnew, axis=1, keepdims=True))
    su_c = jnp.concatenate(su, axis=1)
    sw_c = jnp.concatenate(sw, axis=1)

    @pl.when(j_blk == 0)
    def _init():
        radu_ref[...] = su_c
        radw_ref[...] = sw_c

    @pl.when(j_blk != 0)
    def _acc():
        radu_ref[...] += su_c
        radw_ref[...] += sw_c


def _sc_body(dang_hbm, posx_hbm, posy_hbm, posz_hbm, spin_hbm,
             outd_hbm, outx_hbm, outy_hbm, outz_hbm, outs_hbm,
             posx_v, posy_v, posz_v, spin_v, row_v, cd_v, cj_v,
             std_v, stx_v, sty_v, stz_v, sts_v):
    wid = lax.axis_index("s") * NC + lax.axis_index("c")
    base = wid * ROWS_PER
    pltpu.sync_copy(posx_hbm, posx_v)
    pltpu.sync_copy(posy_hbm, posy_v)
    pltpu.sync_copy(posz_hbm, posz_v)
    pltpu.sync_copy(spin_hbm, spin_v)
    lane_iota = lax.iota(jnp.int32, LANE)
    big_vec = jnp.full((LANE,), BIG, dtype=jnp.float32)
    zero_vec = jnp.zeros((LANE,), dtype=jnp.int32)

    def quad_body(q, _):
        pltpu.sync_copy(dang_hbm.at[pl.ds(base + q * RBATCH, RBATCH)], row_v)
        for rr in range(RBATCH):
            r = q * RBATCH + rr
            for k in range(MSLOT // LANE + 1):
                cd_v[pl.ds(k * LANE, LANE)] = big_vec
                cj_v[pl.ds(k * LANE, LANE)] = zero_vec

            def scan_body(c, cnt):
                v = row_v[rr, pl.ds(c * LANE, LANE)]
                m = v < VTHRESH
                pref = plsc.cumsum(m.astype(jnp.int32))
                slots = cnt + pref - 1
                jvec = c * LANE + lane_iota
                plsc.store_scatter(cd_v, [slots], v, mask=m)
                plsc.store_scatter(cj_v, [slots], jvec, mask=m)
                return cnt + plsc.all_reduce_population_count(m)

            lax.fori_loop(0, N // LANE, scan_body,
                          jnp.zeros((LANE,), jnp.int32), unroll=4)

            for k in range(MSLOT // LANE):
                sl = pl.ds(k * LANE, LANE)
                jv = cj_v[sl]
                std_v[r, sl] = cd_v[sl]
                stx_v[r, sl] = plsc.load_gather(posx_v, [jv])
                sty_v[r, sl] = plsc.load_gather(posy_v, [jv])
                stz_v[r, sl] = plsc.load_gather(posz_v, [jv])
                sts_v[r, sl] = plsc.load_gather(spin_v, [jv])
        return 0

    lax.fori_loop(0, ROWS_PER // RBATCH, quad_body, 0)
    pltpu.sync_copy(std_v, outd_hbm.at[pl.ds(base, ROWS_PER)])
    pltpu.sync_copy(stx_v, outx_hbm.at[pl.ds(base, ROWS_PER)])
    pltpu.sync_copy(sty_v, outy_hbm.at[pl.ds(base, ROWS_PER)])
    pltpu.sync_copy(stz_v, outz_hbm.at[pl.ds(base, ROWS_PER)])
    pltpu.sync_copy(sts_v, outs_hbm.at[pl.ds(base, ROWS_PER)])


_sc_compact = functools.partial(
    pl.kernel,
    out_type=[jax.ShapeDtypeStruct((N, MSLOT), jnp.float32)] * 5,
    mesh=plsc.VectorSubcoreMesh(core_axis_name="c", subcore_axis_name="s"),
    compiler_params=pltpu.CompilerParams(needs_layout_passes=False),
    scratch_types=[
        pltpu.VMEM((N,), jnp.float32),
        pltpu.VMEM((N,), jnp.float32),
        pltpu.VMEM((N,), jnp.float32),
        pltpu.VMEM((N,), jnp.float32),
        pltpu.VMEM((RBATCH, N), jnp.float32),
        pltpu.VMEM((N + LANE,), jnp.float32),
        pltpu.VMEM((N + LANE,), jnp.int32),
        pltpu.VMEM((ROWS_PER, MSLOT), jnp.float32),
        pltpu.VMEM((ROWS_PER, MSLOT), jnp.float32),
        pltpu.VMEM((ROWS_PER, MSLOT), jnp.float32),
        pltpu.VMEM((ROWS_PER, MSLOT), jnp.float32),
        pltpu.VMEM((ROWS_PER, MSLOT), jnp.float32),
    ],
)(_sc_body)


def _ang_kernel(pos_ref, dsel_ref, pxj_ref, pyj_ref, pzj_ref, sj_ref,
                angu_ref, angw_ref):
    xi = pos_ref[:, 0:1]
    yi = pos_ref[:, 1:2]
    zi = pos_ref[:, 2:3]
    dsel = dsel_ref[...]
    valid = dsel < VTHRESH
    da = dsel[:, :, None]
    db = dsel[:, None, :]
    a_sl = jax.lax.broadcasted_iota(jnp.int32, (1, MSLOT, MSLOT), 1)
    b_sl = jax.lax.broadcasted_iota(jnp.int32, (1, MSLOT, MSLOT), 2)
    less = (db < da) | ((db == da) & (b_sl < a_sl))
    rank = jnp.sum((valid[:, None, :] & less).astype(jnp.int32), axis=2)
    keep = valid & (rank < M_ANG)
    rank_k = jnp.where(keep, rank, MSLOT + 1)
    s48 = jax.lax.broadcasted_iota(jnp.int32, (1, MSLOT, M_ANG), 2)
    sel = (rank_k[:, :, None] == s48).astype(jnp.float32)
    cover = jnp.sum(sel, axis=1)
    d48 = jnp.sum(sel * dsel[:, :, None], axis=1) + (1.0 - cover) * BIG
    px48 = jnp.sum(sel * pxj_ref[...][:, :, None], axis=1)
    py48 = jnp.sum(sel * pyj_ref[...][:, :, None], axis=1)
    pz48 = jnp.sum(sel * pzj_ref[...][:, :, None], axis=1)
    s48v = jnp.sum(sel * sj_ref[...][:, :, None], axis=1)
    keep48 = d48 < VTHRESH
    dx = px48 - xi
    dy = py48 - yi
    dz = pz48 - zi
    dn = jnp.sqrt(dx * dx + dy * dy + dz * dz + 1e-12)
    dn_safe = jnp.where(keep48, dn, 1.0)
    inv = 1.0 / dn_safe
    ux = dx * inv
    uy = dy * inv
    uz = dz * inv
    fcn = jnp.where(keep48, 0.5 * (jnp.cos(jnp.float32(np.pi / ANG_CUTOFF) * dn_safe) + 1.0), 0.0)
    fs = fcn * s48v
    C = (ux[:, :, None] * ux[:, None, :]
         + uy[:, :, None] * uy[:, None, :]
         + uz[:, :, None] * uz[:, None, :])
    C = jnp.clip(C, -1.0, 1.0)
    p_sl = jax.lax.broadcasted_iota(jnp.int32, (1, M_ANG, M_ANG), 1)
    q_sl = jax.lax.broadcasted_iota(jnp.int32, (1, M_ANG, M_ANG), 2)
    tri = (p_sl < q_sl).astype(jnp.float32)
    wp = fcn[:, :, None] * fcn[:, None, :] * tri
    swp = fs[:, :, None] * fs[:, None, :] * tri
    su_cols = []
    sw_cols = []
    tkm1 = jnp.ones_like(C)
    tk = C
    for n in range(ANG_ORDER + 1):
        if n == 0:
            T = tkm1
        elif n == 1:
            T = tk
        else:
            tnew = 2.0 * C * tk - tkm1
            tkm1, tk = tk, tnew
            T = tnew
        su_cols.append(jnp.sum(jnp.sum(T * wp, axis=2), axis=1, keepdims=True))
        sw_cols.append(jnp.sum(jnp.sum(T * swp, axis=2), axis=1, keepdims=True))
    angu_ref[...] = jnp.concatenate(su_cols, axis=1)
    angw_ref[...] = jnp.concatenate(sw_cols, axis=1)


def kernel(positions, species_indices):
    pos = positions.astype(jnp.float32)
    spin = (2 * species_indices - 1).astype(jnp.float32)
    posT = pos.T
    spinT = spin[None, :]
    n_rad = RAD_ORDER + 1
    dang = pl.pallas_call(
        _dang_kernel,
        grid=(N // BI, N // BJ),
        in_specs=[
            pl.BlockSpec((BI, 3), lambda i, j: (i, 0)),
            pl.BlockSpec((3, BJ), lambda i, j: (0, j)),
        ],
        out_specs=pl.BlockSpec((BI, BJ), lambda i, j: (i, j)),
        out_shape=jax.ShapeDtypeStruct((N, N), jnp.float32),
        compiler_params=pltpu.CompilerParams(
            dimension_semantics=("parallel", "parallel")),
    )(pos, posT)

    posx = jnp.ravel(posT[0])
    posy = jnp.ravel(posT[1])
    posz = jnp.ravel(posT[2])
    dsel, pxj, pyj, pzj, sj = _sc_compact(dang, posx, posy, posz, spin)

    radu, radw = pl.pallas_call(
        _rad_kernel,
        grid=(N // BI, N // BJ),
        in_specs=[
            pl.BlockSpec((BI, 3), lambda i, j: (i, 0)),
            pl.BlockSpec((3, BJ), lambda i, j: (0, j)),
            pl.BlockSpec((1, BJ), lambda i, j: (0, j)),
        ],
        out_specs=[
            pl.BlockSpec((BI, n_rad), lambda i, j: (i, 0)),
            pl.BlockSpec((BI, n_rad), lambda i, j: (i, 0)),
        ],
        out_shape=[
            jax.ShapeDtypeStruct((N, n_rad), jnp.float32),
            jax.ShapeDtypeStruct((N, n_rad), jnp.float32),
        ],
        compiler_params=pltpu.CompilerParams(
            dimension_semantics=("parallel", "arbitrary")),
    )(pos, posT, spinT)

    n_ang = ANG_ORDER + 1
    angu, angw = pl.pallas_call(
        _ang_kernel,
        grid=(N // BA,),
        in_specs=[
            pl.BlockSpec((BA, 3), lambda i: (i, 0)),
            pl.BlockSpec((BA, MSLOT), lambda i: (i, 0)),
            pl.BlockSpec((BA, MSLOT), lambda i: (i, 0)),
            pl.BlockSpec((BA, MSLOT), lambda i: (i, 0)),
            pl.BlockSpec((BA, MSLOT), lambda i: (i, 0)),
            pl.BlockSpec((BA, MSLOT), lambda i: (i, 0)),
        ],
        out_specs=[
            pl.BlockSpec((BA, n_ang), lambda i: (i, 0)),
            pl.BlockSpec((BA, n_ang), lambda i: (i, 0)),
        ],
        out_shape=[
            jax.ShapeDtypeStruct((N, n_ang), jnp.float32),
            jax.ShapeDtypeStruct((N, n_ang), jnp.float32),
        ],
    )(pos, dsel, pxj, pyj, pzj, sj)

    return jnp.concatenate([radu, radw, angu, angw], axis=1)


# speedup vs baseline: 1.0353x; 1.0353x over previous
import functools

import numpy as np
import jax
from jax import lax
import jax.numpy as jnp
from jax.experimental import pallas as pl
from jax.experimental.pallas import tpu as pltpu
from jax.experimental.pallas import tpu_sc as plsc

N = 4096
RAD_ORDER = 16
RAD_CUTOFF = 6.0
ANG_ORDER = 8
ANG_CUTOFF = 4.0
MIN_CUTOFF = 0.55
M_ANG = 48
BIG = 1e30
VTHRESH = 1e20

BI = 256
BJ = 512
BA = 64
MSLOT = 64

NC = 2
NS = 16
NW = NC * NS
ROWS_PER = N // NW
LANE = 16
RBATCH = 4


def _dang_kernel(pos_ref, posT_ref, dang_ref):
    i_blk = pl.program_id(0)
    j_blk = pl.program_id(1)
    xi = pos_ref[:, 0:1]
    yi = pos_ref[:, 1:2]
    zi = pos_ref[:, 2:3]
    xj = posT_ref[0:1, :]
    yj = posT_ref[1:2, :]
    zj = posT_ref[2:3, :]
    dx = xi - xj
    dy = yi - yj
    dz = zi - zj
    z = dx * dx + dy * dy + dz * dz + 1e-12
    gi = i_blk * BI + jax.lax.broadcasted_iota(jnp.int32, (BI, BJ), 0)
    gj = j_blk * BJ + jax.lax.broadcasted_iota(jnp.int32, (BI, BJ), 1)
    notdiag = gi != gj
    mang = (z <= 16.000001907348633) & (z > 0.30250003933906555) & notdiag
    dang_ref[...] = jnp.where(mang, z, BIG)


def _rad_kernel(pos_ref, posT_ref, spinT_ref, radu_ref, radw_ref):
    i_blk = pl.program_id(0)
    j_blk = pl.program_id(1)
    xi = pos_ref[:, 0:1]
    yi = pos_ref[:, 1:2]
    zi = pos_ref[:, 2:3]
    xj = posT_ref[0:1, :]
    yj = posT_ref[1:2, :]
    zj = posT_ref[2:3, :]
    dx = xi - xj
    dy = yi - yj
    dz = zi - zj
    d = jnp.sqrt(dx * dx + dy * dy + dz * dz + 1e-12)
    gi = i_blk * BI + jax.lax.broadcasted_iota(jnp.int32, (BI, BJ), 0)
    gj = j_blk * BJ + jax.lax.broadcasted_iota(jnp.int32, (BI, BJ), 1)
    notdiag = gi != gj
    mrad = (d <= RAD_CUTOFF) & (d > MIN_CUTOFF) & notdiag
    w = jnp.where(mrad, 0.5 * (jnp.cos(jnp.float32(np.pi / RAD_CUTOFF) * d) + 1.0), 0.0)
    ws = w * spinT_ref[0:1, :]
    x = jnp.where(mrad, 2.0 * (d - MIN_CUTOFF) / (RAD_CUTOFF - MIN_CUTOFF) - 1.0, 0.0)
    su = [jnp.sum(w, axis=1, keepdims=True), jnp.sum(w * x, axis=1, keepdims=True)]
    sw = [jnp.sum(ws, axis=1, keepdims=True), jnp.sum(ws * x, axis=1, keepdims=True)]
    tkm1 = jnp.ones_like(x)
    tk = x
    for _ in range(2, RAD_ORDER + 1):
        tnew = 2.0 * x * tk - tkm1
        tkm1, tk = tk, tnew
        su.append(jnp.sum(w * tnew, axis=1, keepdims=True))
        sw.append(jnp.sum(ws * tnew, axis=1, keepdims=True))
    su_c = jnp.concatenate(su, axis=1)
    sw_c = jnp.concatenate(sw, axis=1)

    @pl.when(j_blk == 0)
    def _init():
        radu_ref[...] = su_c
        radw_ref[...] = sw_c

    @pl.when(j_blk != 0)
    def _acc():
        radu_ref[...] += su_c
        radw_ref[...] += sw_c


def _sc_body(dang_hbm, posx_hbm, posy_hbm, posz_hbm, spin_hbm,
             outd_hbm, outx_hbm, outy_hbm, outz_hbm, outs_hbm,
             posx_v, posy_v, posz_v, spin_v, row_v, cd_v, cj_v,
             std_v, stx_v, sty_v, stz_v, sts_v):
    wid = lax.axis_index("s") * NC + lax.axis_index("c")
    base = wid * ROWS_PER
    pltpu.sync_copy(posx_hbm, posx_v)
    pltpu.sync_copy(posy_hbm, posy_v)
    pltpu.sync_copy(posz_hbm, posz_v)
    pltpu.sync_copy(spin_hbm, spin_v)
    lane_iota = lax.iota(jnp.int32, LANE)
    big_vec = jnp.full((LANE,), BIG, dtype=jnp.float32)
    zero_vec = jnp.zeros((LANE,), dtype=jnp.int32)

    def quad_body(q, _):
        pltpu.sync_copy(dang_hbm.at[pl.ds(base + q * RBATCH, RBATCH)], row_v)
        for rr in range(RBATCH):
            r = q * RBATCH + rr
            for k in range(MSLOT // LANE + 1):
                cd_v[pl.ds(k * LANE, LANE)] = big_vec
                cj_v[pl.ds(k * LANE, LANE)] = zero_vec

            def scan_body(c, cnt):
                v = row_v[rr, pl.ds(c * LANE, LANE)]
                m = v < VTHRESH
                pref = plsc.cumsum(m.astype(jnp.int32))
                slots = cnt + pref - 1
                jvec = c * LANE + lane_iota
                plsc.store_scatter(cd_v, [slots], v, mask=m)
                plsc.store_scatter(cj_v, [slots], jvec, mask=m)
                return cnt + plsc.all_reduce_population_count(m)

            lax.fori_loop(0, N // LANE, scan_body,
                          jnp.zeros((LANE,), jnp.int32), unroll=4)

            for k in range(MSLOT // LANE):
                sl = pl.ds(k * LANE, LANE)
                jv = cj_v[sl]
                std_v[r, sl] = cd_v[sl]
                stx_v[r, sl] = plsc.load_gather(posx_v, [jv])
                sty_v[r, sl] = plsc.load_gather(posy_v, [jv])
                stz_v[r, sl] = plsc.load_gather(posz_v, [jv])
                sts_v[r, sl] = plsc.load_gather(spin_v, [jv])
        return 0

    lax.fori_loop(0, ROWS_PER // RBATCH, quad_body, 0)
    pltpu.sync_copy(std_v, outd_hbm.at[pl.ds(base, ROWS_PER)])
    pltpu.sync_copy(stx_v, outx_hbm.at[pl.ds(base, ROWS_PER)])
    pltpu.sync_copy(sty_v, outy_hbm.at[pl.ds(base, ROWS_PER)])
    pltpu.sync_copy(stz_v, outz_hbm.at[pl.ds(base, ROWS_PER)])
    pltpu.sync_copy(sts_v, outs_hbm.at[pl.ds(base, ROWS_PER)])


_sc_compact = functools.partial(
    pl.kernel,
    out_type=[jax.ShapeDtypeStruct((N, MSLOT), jnp.float32)] * 5,
    mesh=plsc.VectorSubcoreMesh(core_axis_name="c", subcore_axis_name="s"),
    compiler_params=pltpu.CompilerParams(needs_layout_passes=False),
    scratch_types=[
        pltpu.VMEM((N,), jnp.float32),
        pltpu.VMEM((N,), jnp.float32),
        pltpu.VMEM((N,), jnp.float32),
        pltpu.VMEM((N,), jnp.float32),
        pltpu.VMEM((RBATCH, N), jnp.float32),
        pltpu.VMEM((N + LANE,), jnp.float32),
        pltpu.VMEM((N + LANE,), jnp.int32),
        pltpu.VMEM((ROWS_PER, MSLOT), jnp.float32),
        pltpu.VMEM((ROWS_PER, MSLOT), jnp.float32),
        pltpu.VMEM((ROWS_PER, MSLOT), jnp.float32),
        pltpu.VMEM((ROWS_PER, MSLOT), jnp.float32),
        pltpu.VMEM((ROWS_PER, MSLOT), jnp.float32),
    ],
)(_sc_body)


def _ang_kernel(pos_ref, dsel_ref, pxj_ref, pyj_ref, pzj_ref, sj_ref,
                angu_ref, angw_ref):
    xi = pos_ref[:, 0:1]
    yi = pos_ref[:, 1:2]
    zi = pos_ref[:, 2:3]
    dsel = dsel_ref[...]
    valid = dsel < VTHRESH
    da = dsel[:, :, None]
    db = dsel[:, None, :]
    a_sl = jax.lax.broadcasted_iota(jnp.int32, (1, MSLOT, MSLOT), 1)
    b_sl = jax.lax.broadcasted_iota(jnp.int32, (1, MSLOT, MSLOT), 2)
    less = (db < da) | ((db == da) & (b_sl < a_sl))
    rank = jnp.sum((valid[:, None, :] & less).astype(jnp.int32), axis=2)
    keep = valid & (rank < M_ANG)
    dx = pxj_ref[...] - xi
    dy = pyj_ref[...] - yi
    dz = pzj_ref[...] - zi
    dn = jnp.sqrt(dx * dx + dy * dy + dz * dz + 1e-12)
    dn_safe = jnp.where(keep, dn, 1.0)
    inv = 1.0 / dn_safe
    ux = dx * inv
    uy = dy * inv
    uz = dz * inv
    fcn = jnp.where(keep, 0.5 * (jnp.cos(jnp.float32(np.pi / ANG_CUTOFF) * dn_safe) + 1.0), 0.0)
    fs = fcn * sj_ref[...]
    C = (ux[:, :, None] * ux[:, None, :]
         + uy[:, :, None] * uy[:, None, :]
         + uz[:, :, None] * uz[:, None, :])
    C = jnp.clip(C, -1.0, 1.0)
    tri = (a_sl < b_sl).astype(jnp.float32)
    wp = fcn[:, :, None] * fcn[:, None, :] * tri
    swp = fs[:, :, None] * fs[:, None, :] * tri
    su_cols = []
    sw_cols = []
    tkm1 = jnp.ones_like(C)
    tk = C
    for n in range(ANG_ORDER + 1):
        if n == 0:
            T = tkm1
        elif n == 1:
            T = tk
        else:
            tnew = 2.0 * C * tk - tkm1
            tkm1, tk = tk, tnew
            T = tnew
        su_cols.append(jnp.sum(jnp.sum(T * wp, axis=2), axis=1, keepdims=True))
        sw_cols.append(jnp.sum(jnp.sum(T * swp, axis=2), axis=1, keepdims=True))
    angu_ref[...] = jnp.concatenate(su_cols, axis=1)
    angw_ref[...] = jnp.concatenate(sw_cols, axis=1)


def kernel(positions, species_indices):
    pos = positions.astype(jnp.float32)
    spin = (2 * species_indices - 1).astype(jnp.float32)
    posT = pos.T
    spinT = spin[None, :]
    n_rad = RAD_ORDER + 1
    dang = pl.pallas_call(
        _dang_kernel,
        grid=(N // BI, N // BJ),
        in_specs=[
            pl.BlockSpec((BI, 3), lambda i, j: (i, 0)),
            pl.BlockSpec((3, BJ), lambda i, j: (0, j)),
        ],
        out_specs=pl.BlockSpec((BI, BJ), lambda i, j: (i, j)),
        out_shape=jax.ShapeDtypeStruct((N, N), jnp.float32),
        compiler_params=pltpu.CompilerParams(
            dimension_semantics=("parallel", "parallel")),
    )(pos, posT)

    posx = jnp.ravel(posT[0])
    posy = jnp.ravel(posT[1])
    posz = jnp.ravel(posT[2])
    dsel, pxj, pyj, pzj, sj = _sc_compact(dang, posx, posy, posz, spin)

    radu, radw = pl.pallas_call(
        _rad_kernel,
        grid=(N // BI, N // BJ),
        in_specs=[
            pl.BlockSpec((BI, 3), lambda i, j: (i, 0)),
            pl.BlockSpec((3, BJ), lambda i, j: (0, j)),
            pl.BlockSpec((1, BJ), lambda i, j: (0, j)),
        ],
        out_specs=[
            pl.BlockSpec((BI, n_rad), lambda i, j: (i, 0)),
            pl.BlockSpec((BI, n_rad), lambda i, j: (i, 0)),
        ],
        out_shape=[
            jax.ShapeDtypeStruct((N, n_rad), jnp.float32),
            jax.ShapeDtypeStruct((N, n_rad), jnp.float32),
        ],
        compiler_params=pltpu.CompilerParams(
            dimension_semantics=("parallel", "arbitrary")),
    )(pos, posT, spinT)

    n_ang = ANG_ORDER + 1
    angu, angw = pl.pallas_call(
        _ang_kernel,
        grid=(N // BA,),
        in_specs=[
            pl.BlockSpec((BA, 3), lambda i: (i, 0)),
            pl.BlockSpec((BA, MSLOT), lambda i: (i, 0)),
            pl.BlockSpec((BA, MSLOT), lambda i: (i, 0)),
            pl.BlockSpec((BA, MSLOT), lambda i: (i, 0)),
            pl.BlockSpec((BA, MSLOT), lambda i: (i, 0)),
            pl.BlockSpec((BA, MSLOT), lambda i: (i, 0)),
        ],
        out_specs=[
            pl.BlockSpec((BA, n_ang), lambda i: (i, 0)),
            pl.BlockSpec((BA, n_ang), lambda i: (i, 0)),
        ],
        out_shape=[
            jax.ShapeDtypeStruct((N, n_ang), jnp.float32),
            jax.ShapeDtypeStruct((N, n_ang), jnp.float32),
        ],
    )(pos, dsel, pxj, pyj, pzj, sj)

    return jnp.concatenate([radu, radw, angu, angw], axis=1)


# speedup vs baseline: 1.7287x; 1.6698x over previous
import functools

import numpy as np
import jax
from jax import lax
import jax.numpy as jnp
from jax.experimental import pallas as pl
from jax.experimental.pallas import tpu as pltpu
from jax.experimental.pallas import tpu_sc as plsc

N = 4096
RAD_ORDER = 16
RAD_CUTOFF = 6.0
ANG_ORDER = 8
ANG_CUTOFF = 4.0
MIN_CUTOFF = 0.55
M_ANG = 48
BIG = 1e30
VTHRESH = 1e20


def _cheb_monomial_coeffs(nmax):
    polys = [[1], [0, 1]]
    for n in range(2, nmax + 1):
        c = [0] * (n + 1)
        for i, v in enumerate(polys[n - 1]):
            c[i + 1] += 2 * v
        for i, v in enumerate(polys[n - 2]):
            c[i] -= v
        polys.append(c)
    return polys


_CHEB = _cheb_monomial_coeffs(ANG_ORDER)
_FACT = [1, 1, 2, 6, 24, 120, 720, 5040, 40320]

BI = 256
BJ = 512
BA = 64
MSLOT = 64

NC = 2
NS = 16
NW = NC * NS
ROWS_PER = N // NW
LANE = 16
RBATCH = 4


def _dang_kernel(pos_ref, posT_ref, dang_ref):
    i_blk = pl.program_id(0)
    j_blk = pl.program_id(1)
    xi = pos_ref[:, 0:1]
    yi = pos_ref[:, 1:2]
    zi = pos_ref[:, 2:3]
    xj = posT_ref[0:1, :]
    yj = posT_ref[1:2, :]
    zj = posT_ref[2:3, :]
    dx = xi - xj
    dy = yi - yj
    dz = zi - zj
    z = dx * dx + dy * dy + dz * dz + 1e-12
    gi = i_blk * BI + jax.lax.broadcasted_iota(jnp.int32, (BI, BJ), 0)
    gj = j_blk * BJ + jax.lax.broadcasted_iota(jnp.int32, (BI, BJ), 1)
    notdiag = gi != gj
    mang = (z <= 16.000001907348633) & (z > 0.30250003933906555) & notdiag
    dang_ref[...] = jnp.where(mang, z, BIG)


def _rad_kernel(pos_ref, posT_ref, spinT_ref, radu_ref, radw_ref):
    i_blk = pl.program_id(0)
    j_blk = pl.program_id(1)
    xi = pos_ref[:, 0:1]
    yi = pos_ref[:, 1:2]
    zi = pos_ref[:, 2:3]
    xj = posT_ref[0:1, :]
    yj = posT_ref[1:2, :]
    zj = posT_ref[2:3, :]
    dx = xi - xj
    dy = yi - yj
    dz = zi - zj
    d = jnp.sqrt(dx * dx + dy * dy + dz * dz + 1e-12)
    gi = i_blk * BI + jax.lax.broadcasted_iota(jnp.int32, (BI, BJ), 0)
    gj = j_blk * BJ + jax.lax.broadcasted_iota(jnp.int32, (BI, BJ), 1)
    notdiag = gi != gj
    mrad = (d <= RAD_CUTOFF) & (d > MIN_CUTOFF) & notdiag
    w = jnp.where(mrad, 0.5 * (jnp.cos(jnp.float32(np.pi / RAD_CUTOFF) * d) + 1.0), 0.0)
    ws = w * spinT_ref[0:1, :]
    x = jnp.where(mrad, 2.0 * (d - MIN_CUTOFF) / (RAD_CUTOFF - MIN_CUTOFF) - 1.0, 0.0)
    su = [jnp.sum(w, axis=1, keepdims=True), jnp.sum(w * x, axis=1, keepdims=True)]
    sw = [jnp.sum(ws, axis=1, keepdims=True), jnp.sum(ws * x, axis=1, keepdims=True)]
    tkm1 = jnp.ones_like(x)
    tk = x
    for _ in range(2, RAD_ORDER + 1):
        tnew = 2.0 * x * tk - tkm1
        tkm1, tk = tk, tnew
        su.append(jnp.sum(w * tnew, axis=1, keepdims=True))
        sw.append(jnp.sum(ws * tnew, axis=1, keepdims=True))
    su_c = jnp.concatenate(su, axis=1)
    sw_c = jnp.concatenate(sw, axis=1)

    @pl.when(j_blk == 0)
    def _init():
        radu_ref[...] = su_c
        radw_ref[...] = sw_c

    @pl.when(j_blk != 0)
    def _acc():
        radu_ref[...] += su_c
        radw_ref[...] += sw_c


def _sc_body(dang_hbm, posx_hbm, posy_hbm, posz_hbm, spin_hbm,
             outd_hbm, outx_hbm, outy_hbm, outz_hbm, outs_hbm,
             posx_v, posy_v, posz_v, spin_v, row_v, cd_v, cj_v,
             std_v, stx_v, sty_v, stz_v, sts_v):
    wid = lax.axis_index("s") * NC + lax.axis_index("c")
    base = wid * ROWS_PER
    pltpu.sync_copy(posx_hbm, posx_v)
    pltpu.sync_copy(posy_hbm, posy_v)
    pltpu.sync_copy(posz_hbm, posz_v)
    pltpu.sync_copy(spin_hbm, spin_v)
    lane_iota = lax.iota(jnp.int32, LANE)
    big_vec = jnp.full((LANE,), BIG, dtype=jnp.float32)
    zero_vec = jnp.zeros((LANE,), dtype=jnp.int32)

    def quad_body(q, _):
        pltpu.sync_copy(dang_hbm.at[pl.ds(base + q * RBATCH, RBATCH)], row_v)
        for rr in range(RBATCH):
            r = q * RBATCH + rr
            for k in range(MSLOT // LANE + 1):
                cd_v[pl.ds(k * LANE, LANE)] = big_vec
                cj_v[pl.ds(k * LANE, LANE)] = zero_vec

            def scan_body(c, cnt):
                v = row_v[rr, pl.ds(c * LANE, LANE)]
                m = v < VTHRESH
                pref = plsc.cumsum(m.astype(jnp.int32))
                slots = cnt + pref - 1
                jvec = c * LANE + lane_iota
                plsc.store_scatter(cd_v, [slots], v, mask=m)
                plsc.store_scatter(cj_v, [slots], jvec, mask=m)
                return cnt + plsc.all_reduce_population_count(m)

            lax.fori_loop(0, N // LANE, scan_body,
                          jnp.zeros((LANE,), jnp.int32), unroll=4)

            for k in range(MSLOT // LANE):
                sl = pl.ds(k * LANE, LANE)
                jv = cj_v[sl]
                std_v[r, sl] = cd_v[sl]
                stx_v[r, sl] = plsc.load_gather(posx_v, [jv])
                sty_v[r, sl] = plsc.load_gather(posy_v, [jv])
                stz_v[r, sl] = plsc.load_gather(posz_v, [jv])
                sts_v[r, sl] = plsc.load_gather(spin_v, [jv])
        return 0

    lax.fori_loop(0, ROWS_PER // RBATCH, quad_body, 0)
    pltpu.sync_copy(std_v, outd_hbm.at[pl.ds(base, ROWS_PER)])
    pltpu.sync_copy(stx_v, outx_hbm.at[pl.ds(base, ROWS_PER)])
    pltpu.sync_copy(sty_v, outy_hbm.at[pl.ds(base, ROWS_PER)])
    pltpu.sync_copy(stz_v, outz_hbm.at[pl.ds(base, ROWS_PER)])
    pltpu.sync_copy(sts_v, outs_hbm.at[pl.ds(base, ROWS_PER)])


_sc_compact = functools.partial(
    pl.kernel,
    out_type=[jax.ShapeDtypeStruct((N, MSLOT), jnp.float32)] * 5,
    mesh=plsc.VectorSubcoreMesh(core_axis_name="c", subcore_axis_name="s"),
    compiler_params=pltpu.CompilerParams(needs_layout_passes=False),
    scratch_types=[
        pltpu.VMEM((N,), jnp.float32),
        pltpu.VMEM((N,), jnp.float32),
        pltpu.VMEM((N,), jnp.float32),
        pltpu.VMEM((N,), jnp.float32),
        pltpu.VMEM((RBATCH, N), jnp.float32),
        pltpu.VMEM((N + LANE,), jnp.float32),
        pltpu.VMEM((N + LANE,), jnp.int32),
        pltpu.VMEM((ROWS_PER, MSLOT), jnp.float32),
        pltpu.VMEM((ROWS_PER, MSLOT), jnp.float32),
        pltpu.VMEM((ROWS_PER, MSLOT), jnp.float32),
        pltpu.VMEM((ROWS_PER, MSLOT), jnp.float32),
        pltpu.VMEM((ROWS_PER, MSLOT), jnp.float32),
    ],
)(_sc_body)


def _ang_kernel(pos_ref, dsel_ref, pxj_ref, pyj_ref, pzj_ref, sj_ref,
                angu_ref, angw_ref):
    xi = pos_ref[:, 0:1]
    yi = pos_ref[:, 1:2]
    zi = pos_ref[:, 2:3]
    dsel = dsel_ref[...]
    valid = dsel < VTHRESH
    da = dsel[:, :, None]
    db = dsel[:, None, :]
    a_sl = jax.lax.broadcasted_iota(jnp.int32, (1, MSLOT, MSLOT), 1)
    b_sl = jax.lax.broadcasted_iota(jnp.int32, (1, MSLOT, MSLOT), 2)
    less = (db < da) | ((db == da) & (b_sl < a_sl))
    rank = jnp.sum((valid[:, None, :] & less).astype(jnp.int32), axis=2)
    keep = valid & (rank < M_ANG)
    dx = pxj_ref[...] - xi
    dy = pyj_ref[...] - yi
    dz = pzj_ref[...] - zi
    dn = jnp.sqrt(dx * dx + dy * dy + dz * dz + 1e-12)
    dn_safe = jnp.where(keep, dn, 1.0)
    inv = 1.0 / dn_safe
    ux = dx * inv
    uy = dy * inv
    uz = dz * inv
    fcn = jnp.where(keep, 0.5 * (jnp.cos(jnp.float32(np.pi / ANG_CUTOFF) * dn_safe) + 1.0), 0.0)
    fs = fcn * sj_ref[...]
    ux = jnp.where(keep, ux, 0.0)
    uy = jnp.where(keep, uy, 0.0)
    uz = jnp.where(keep, uz, 0.0)
    Sf = []
    Sg = []
    cur = {(0, 0, 0): None}
    for p in range(ANG_ORDER + 1):
        if p > 0:
            nxt = {}
            for a in range(p, -1, -1):
                for b in range(p - a, -1, -1):
                    c = p - a - b
                    if a > 0:
                        parent, fac = (a - 1, b, c), ux
                    elif b > 0:
                        parent, fac = (a, b - 1, c), uy
                    else:
                        parent, fac = (a, b, c - 1), uz
                    pv = cur[parent]
                    nxt[(a, b, c)] = fac if pv is None else pv * fac
            cur = nxt
        sf_p = None
        sg_p = None
        for (a, b, c), val in cur.items():
            mult = float(_FACT[p] // (_FACT[a] * _FACT[b] * _FACT[c]))
            wf = fcn if val is None else fcn * val
            wg = fs if val is None else fs * val
            mf = jnp.sum(wf, axis=1, keepdims=True)
            mg = jnp.sum(wg, axis=1, keepdims=True)
            tf = mult * (mf * mf)
            tg = mult * (mg * mg)
            sf_p = tf if sf_p is None else sf_p + tf
            sg_p = tg if sg_p is None else sg_p + tg
        Sf.append(sf_p)
        Sg.append(sg_p)
    sumf2 = jnp.sum(fcn * fcn, axis=1, keepdims=True)
    su_cols = []
    sw_cols = []
    for n in range(ANG_ORDER + 1):
        tot_f = None
        tot_g = None
        for p, cc in enumerate(_CHEB[n]):
            if cc == 0:
                continue
            tf = float(cc) * Sf[p]
            tg = float(cc) * Sg[p]
            tot_f = tf if tot_f is None else tot_f + tf
            tot_g = tg if tot_g is None else tot_g + tg
        su_cols.append(0.5 * (tot_f - sumf2))
        sw_cols.append(0.5 * (tot_g - sumf2))
    angu_ref[...] = jnp.concatenate(su_cols, axis=1)
    angw_ref[...] = jnp.concatenate(sw_cols, axis=1)


def kernel(positions, species_indices):
    pos = positions.astype(jnp.float32)
    spin = (2 * species_indices - 1).astype(jnp.float32)
    posT = pos.T
    spinT = spin[None, :]
    n_rad = RAD_ORDER + 1
    dang = pl.pallas_call(
        _dang_kernel,
        grid=(N // BI, N // BJ),
        in_specs=[
            pl.BlockSpec((BI, 3), lambda i, j: (i, 0)),
            pl.BlockSpec((3, BJ), lambda i, j: (0, j)),
        ],
        out_specs=pl.BlockSpec((BI, BJ), lambda i, j: (i, j)),
        out_shape=jax.ShapeDtypeStruct((N, N), jnp.float32),
        compiler_params=pltpu.CompilerParams(
            dimension_semantics=("parallel", "parallel")),
    )(pos, posT)

    posx = jnp.ravel(posT[0])
    posy = jnp.ravel(posT[1])
    posz = jnp.ravel(posT[2])
    dsel, pxj, pyj, pzj, sj = _sc_compact(dang, posx, posy, posz, spin)

    radu, radw = pl.pallas_call(
        _rad_kernel,
        grid=(N // BI, N // BJ),
        in_specs=[
            pl.BlockSpec((BI, 3), lambda i, j: (i, 0)),
            pl.BlockSpec((3, BJ), lambda i, j: (0, j)),
            pl.BlockSpec((1, BJ), lambda i, j: (0, j)),
        ],
        out_specs=[
            pl.BlockSpec((BI, n_rad), lambda i, j: (i, 0)),
            pl.BlockSpec((BI, n_rad), lambda i, j: (i, 0)),
        ],
        out_shape=[
            jax.ShapeDtypeStruct((N, n_rad), jnp.float32),
            jax.ShapeDtypeStruct((N, n_rad), jnp.float32),
        ],
        compiler_params=pltpu.CompilerParams(
            dimension_semantics=("parallel", "arbitrary")),
    )(pos, posT, spinT)

    n_ang = ANG_ORDER + 1
    angu, angw = pl.pallas_call(
        _ang_kernel,
        grid=(N // BA,),
        in_specs=[
            pl.BlockSpec((BA, 3), lambda i: (i, 0)),
            pl.BlockSpec((BA, MSLOT), lambda i: (i, 0)),
            pl.BlockSpec((BA, MSLOT), lambda i: (i, 0)),
            pl.BlockSpec((BA, MSLOT), lambda i: (i, 0)),
            pl.BlockSpec((BA, MSLOT), lambda i: (i, 0)),
            pl.BlockSpec((BA, MSLOT), lambda i: (i, 0)),
        ],
        out_specs=[
            pl.BlockSpec((BA, n_ang), lambda i: (i, 0)),
            pl.BlockSpec((BA, n_ang), lambda i: (i, 0)),
        ],
        out_shape=[
            jax.ShapeDtypeStruct((N, n_ang), jnp.float32),
            jax.ShapeDtypeStruct((N, n_ang), jnp.float32),
        ],
    )(pos, dsel, pxj, pyj, pzj, sj)

    return jnp.concatenate([radu, radw, angu, angw], axis=1)


# speedup vs baseline: 1.7327x; 1.0023x over previous
import functools

import numpy as np
import jax
from jax import lax
import jax.numpy as jnp
from jax.experimental import pallas as pl
from jax.experimental.pallas import tpu as pltpu
from jax.experimental.pallas import tpu_sc as plsc

N = 4096
RAD_ORDER = 16
RAD_CUTOFF = 6.0
ANG_ORDER = 8
ANG_CUTOFF = 4.0
MIN_CUTOFF = 0.55
M_ANG = 48
BIG = 1e30
VTHRESH = 1e20


def _cheb_monomial_coeffs(nmax):
    polys = [[1], [0, 1]]
    for n in range(2, nmax + 1):
        c = [0] * (n + 1)
        for i, v in enumerate(polys[n - 1]):
            c[i + 1] += 2 * v
        for i, v in enumerate(polys[n - 2]):
            c[i] -= v
        polys.append(c)
    return polys


_CHEB = _cheb_monomial_coeffs(ANG_ORDER)
_FACT = [1, 1, 2, 6, 24, 120, 720, 5040, 40320]

BI = 256
BJ = 512
BA = 64
MSLOT = 64

NC = 2
NS = 16
NW = NC * NS
ROWS_PER = N // NW
LANE = 16
RBATCH = 4


def _dang_kernel(pos_ref, posT_ref, dang_ref):
    xi = pos_ref[:, 0:1]
    yi = pos_ref[:, 1:2]
    zi = pos_ref[:, 2:3]
    xj = posT_ref[0:1, :]
    yj = posT_ref[1:2, :]
    zj = posT_ref[2:3, :]
    dx = xi - xj
    dy = yi - yj
    dz = zi - zj
    z = dx * dx + dy * dy + dz * dz + 1e-12
    mang = (z <= 16.000001907348633) & (z > 0.30250003933906555)
    dang_ref[...] = jnp.where(mang, z, BIG)


def _rad_kernel(pos_ref, posT_ref, spinT_ref, radu_ref, radw_ref):
    j_blk = pl.program_id(1)
    xi = pos_ref[:, 0:1]
    yi = pos_ref[:, 1:2]
    zi = pos_ref[:, 2:3]
    xj = posT_ref[0:1, :]
    yj = posT_ref[1:2, :]
    zj = posT_ref[2:3, :]
    dx = xi - xj
    dy = yi - yj
    dz = zi - zj
    d = jnp.sqrt(dx * dx + dy * dy + dz * dz + 1e-12)
    mrad = (d <= RAD_CUTOFF) & (d > MIN_CUTOFF)
    w = jnp.where(mrad, 0.5 * (jnp.cos(jnp.float32(np.pi / RAD_CUTOFF) * d) + 1.0), 0.0)
    ws = w * spinT_ref[0:1, :]
    x = jnp.where(mrad, 2.0 * (d - MIN_CUTOFF) / (RAD_CUTOFF - MIN_CUTOFF) - 1.0, 0.0)
    su = [jnp.sum(w, axis=1, keepdims=True), jnp.sum(w * x, axis=1, keepdims=True)]
    sw = [jnp.sum(ws, axis=1, keepdims=True), jnp.sum(ws * x, axis=1, keepdims=True)]
    tkm1 = jnp.ones_like(x)
    tk = x
    for _ in range(2, RAD_ORDER + 1):
        tnew = 2.0 * x * tk - tkm1
        tkm1, tk = tk, tnew
        su.append(jnp.sum(w * tnew, axis=1, keepdims=True))
        sw.append(jnp.sum(ws * tnew, axis=1, keepdims=True))
    su_c = jnp.concatenate(su, axis=1)
    sw_c = jnp.concatenate(sw, axis=1)

    @pl.when(j_blk == 0)
    def _init():
        radu_ref[...] = su_c
        radw_ref[...] = sw_c

    @pl.when(j_blk != 0)
    def _acc():
        radu_ref[...] += su_c
        radw_ref[...] += sw_c


def _sc_body(dang_hbm, posx_hbm, posy_hbm, posz_hbm, spin_hbm,
             outd_hbm, outx_hbm, outy_hbm, outz_hbm, outs_hbm,
             posx_v, posy_v, posz_v, spin_v, row_v, cd_v, cj_v,
             std_v, stx_v, sty_v, stz_v, sts_v):
    wid = lax.axis_index("s") * NC + lax.axis_index("c")
    base = wid * ROWS_PER
    pltpu.sync_copy(posx_hbm, posx_v)
    pltpu.sync_copy(posy_hbm, posy_v)
    pltpu.sync_copy(posz_hbm, posz_v)
    pltpu.sync_copy(spin_hbm, spin_v)
    lane_iota = lax.iota(jnp.int32, LANE)
    big_vec = jnp.full((LANE,), BIG, dtype=jnp.float32)
    zero_vec = jnp.zeros((LANE,), dtype=jnp.int32)

    def quad_body(q, _):
        pltpu.sync_copy(dang_hbm.at[pl.ds(base + q * RBATCH, RBATCH)], row_v)
        for rr in range(RBATCH):
            r = q * RBATCH + rr
            for k in range(MSLOT // LANE + 1):
                cd_v[pl.ds(k * LANE, LANE)] = big_vec
                cj_v[pl.ds(k * LANE, LANE)] = zero_vec

            def scan_body(c, cnt):
                v = row_v[rr, pl.ds(c * LANE, LANE)]
                m = v < VTHRESH
                pref = plsc.cumsum(m.astype(jnp.int32))
                slots = cnt + pref - 1
                jvec = c * LANE + lane_iota
                plsc.store_scatter(cd_v, [slots], v, mask=m)
                plsc.store_scatter(cj_v, [slots], jvec, mask=m)
                return cnt + plsc.all_reduce_population_count(m)

            lax.fori_loop(0, N // LANE, scan_body,
                          jnp.zeros((LANE,), jnp.int32), unroll=4)

            for k in range(MSLOT // LANE):
                sl = pl.ds(k * LANE, LANE)
                jv = cj_v[sl]
                std_v[r, sl] = cd_v[sl]
                stx_v[r, sl] = plsc.load_gather(posx_v, [jv])
                sty_v[r, sl] = plsc.load_gather(posy_v, [jv])
                stz_v[r, sl] = plsc.load_gather(posz_v, [jv])
                sts_v[r, sl] = plsc.load_gather(spin_v, [jv])
        return 0

    lax.fori_loop(0, ROWS_PER // RBATCH, quad_body, 0)
    pltpu.sync_copy(std_v, outd_hbm.at[pl.ds(base, ROWS_PER)])
    pltpu.sync_copy(stx_v, outx_hbm.at[pl.ds(base, ROWS_PER)])
    pltpu.sync_copy(sty_v, outy_hbm.at[pl.ds(base, ROWS_PER)])
    pltpu.sync_copy(stz_v, outz_hbm.at[pl.ds(base, ROWS_PER)])
    pltpu.sync_copy(sts_v, outs_hbm.at[pl.ds(base, ROWS_PER)])


_sc_compact = functools.partial(
    pl.kernel,
    out_type=[jax.ShapeDtypeStruct((N, MSLOT), jnp.float32)] * 5,
    mesh=plsc.VectorSubcoreMesh(core_axis_name="c", subcore_axis_name="s"),
    compiler_params=pltpu.CompilerParams(needs_layout_passes=False),
    scratch_types=[
        pltpu.VMEM((N,), jnp.float32),
        pltpu.VMEM((N,), jnp.float32),
        pltpu.VMEM((N,), jnp.float32),
        pltpu.VMEM((N,), jnp.float32),
        pltpu.VMEM((RBATCH, N), jnp.float32),
        pltpu.VMEM((N + LANE,), jnp.float32),
        pltpu.VMEM((N + LANE,), jnp.int32),
        pltpu.VMEM((ROWS_PER, MSLOT), jnp.float32),
        pltpu.VMEM((ROWS_PER, MSLOT), jnp.float32),
        pltpu.VMEM((ROWS_PER, MSLOT), jnp.float32),
        pltpu.VMEM((ROWS_PER, MSLOT), jnp.float32),
        pltpu.VMEM((ROWS_PER, MSLOT), jnp.float32),
    ],
)(_sc_body)


def _ang_kernel(pos_ref, dsel_ref, pxj_ref, pyj_ref, pzj_ref, sj_ref,
                angu_ref, angw_ref):
    xi = pos_ref[:, 0:1]
    yi = pos_ref[:, 1:2]
    zi = pos_ref[:, 2:3]
    dsel = dsel_ref[...]
    valid = dsel < VTHRESH
    da = dsel[:, :, None]
    db = dsel[:, None, :]
    a_sl = jax.lax.broadcasted_iota(jnp.int32, (1, MSLOT, MSLOT), 1)
    b_sl = jax.lax.broadcasted_iota(jnp.int32, (1, MSLOT, MSLOT), 2)
    less = (db < da) | ((db == da) & (b_sl < a_sl))
    rank = jnp.sum((valid[:, None, :] & less).astype(jnp.int32), axis=2)
    keep = valid & (rank < M_ANG)
    dx = pxj_ref[...] - xi
    dy = pyj_ref[...] - yi
    dz = pzj_ref[...] - zi
    dn = jnp.sqrt(dx * dx + dy * dy + dz * dz + 1e-12)
    dn_safe = jnp.where(keep, dn, 1.0)
    inv = 1.0 / dn_safe
    ux = dx * inv
    uy = dy * inv
    uz = dz * inv
    fcn = jnp.where(keep, 0.5 * (jnp.cos(jnp.float32(np.pi / ANG_CUTOFF) * dn_safe) + 1.0), 0.0)
    fs = fcn * sj_ref[...]
    ux = jnp.where(keep, ux, 0.0)
    uy = jnp.where(keep, uy, 0.0)
    uz = jnp.where(keep, uz, 0.0)
    Sf = []
    Sg = []
    cur = {(0, 0, 0): None}
    for p in range(ANG_ORDER + 1):
        if p > 0:
            nxt = {}
            for a in range(p, -1, -1):
                for b in range(p - a, -1, -1):
                    c = p - a - b
                    if a > 0:
                        parent, fac = (a - 1, b, c), ux
                    elif b > 0:
                        parent, fac = (a, b - 1, c), uy
                    else:
                        parent, fac = (a, b, c - 1), uz
                    pv = cur[parent]
                    nxt[(a, b, c)] = fac if pv is None else pv * fac
            cur = nxt
        sf_p = None
        sg_p = None
        for (a, b, c), val in cur.items():
            mult = float(_FACT[p] // (_FACT[a] * _FACT[b] * _FACT[c]))
            wf = fcn if val is None else fcn * val
            wg = fs if val is None else fs * val
            mf = jnp.sum(wf, axis=1, keepdims=True)
            mg = jnp.sum(wg, axis=1, keepdims=True)
            tf = mult * (mf * mf)
            tg = mult * (mg * mg)
            sf_p = tf if sf_p is None else sf_p + tf
            sg_p = tg if sg_p is None else sg_p + tg
        Sf.append(sf_p)
        Sg.append(sg_p)
    sumf2 = jnp.sum(fcn * fcn, axis=1, keepdims=True)
    su_cols = []
    sw_cols = []
    for n in range(ANG_ORDER + 1):
        tot_f = None
        tot_g = None
        for p, cc in enumerate(_CHEB[n]):
            if cc == 0:
                continue
            tf = float(cc) * Sf[p]
            tg = float(cc) * Sg[p]
            tot_f = tf if tot_f is None else tot_f + tf
            tot_g = tg if tot_g is None else tot_g + tg
        su_cols.append(0.5 * (tot_f - sumf2))
        sw_cols.append(0.5 * (tot_g - sumf2))
    angu_ref[...] = jnp.concatenate(su_cols, axis=1)
    angw_ref[...] = jnp.concatenate(sw_cols, axis=1)


def kernel(positions, species_indices):
    pos = positions.astype(jnp.float32)
    spin = (2 * species_indices - 1).astype(jnp.float32)
    posT = pos.T
    spinT = spin[None, :]
    n_rad = RAD_ORDER + 1
    dang = pl.pallas_call(
        _dang_kernel,
        grid=(N // BI, N // BJ),
        in_specs=[
            pl.BlockSpec((BI, 3), lambda i, j: (i, 0)),
            pl.BlockSpec((3, BJ), lambda i, j: (0, j)),
        ],
        out_specs=pl.BlockSpec((BI, BJ), lambda i, j: (i, j)),
        out_shape=jax.ShapeDtypeStruct((N, N), jnp.float32),
        compiler_params=pltpu.CompilerParams(
            dimension_semantics=("parallel", "parallel")),
    )(pos, posT)

    posx = jnp.ravel(posT[0])
    posy = jnp.ravel(posT[1])
    posz = jnp.ravel(posT[2])
    dsel, pxj, pyj, pzj, sj = _sc_compact(dang, posx, posy, posz, spin)

    radu, radw = pl.pallas_call(
        _rad_kernel,
        grid=(N // BI, N // BJ),
        in_specs=[
            pl.BlockSpec((BI, 3), lambda i, j: (i, 0)),
            pl.BlockSpec((3, BJ), lambda i, j: (0, j)),
            pl.BlockSpec((1, BJ), lambda i, j: (0, j)),
        ],
        out_specs=[
            pl.BlockSpec((BI, n_rad), lambda i, j: (i, 0)),
            pl.BlockSpec((BI, n_rad), lambda i, j: (i, 0)),
        ],
        out_shape=[
            jax.ShapeDtypeStruct((N, n_rad), jnp.float32),
            jax.ShapeDtypeStruct((N, n_rad), jnp.float32),
        ],
        compiler_params=pltpu.CompilerParams(
            dimension_semantics=("parallel", "arbitrary")),
    )(pos, posT, spinT)

    n_ang = ANG_ORDER + 1
    angu, angw = pl.pallas_call(
        _ang_kernel,
        grid=(N // BA,),
        in_specs=[
            pl.BlockSpec((BA, 3), lambda i: (i, 0)),
            pl.BlockSpec((BA, MSLOT), lambda i: (i, 0)),
            pl.BlockSpec((BA, MSLOT), lambda i: (i, 0)),
            pl.BlockSpec((BA, MSLOT), lambda i: (i, 0)),
            pl.BlockSpec((BA, MSLOT), lambda i: (i, 0)),
            pl.BlockSpec((BA, MSLOT), lambda i: (i, 0)),
        ],
        out_specs=[
            pl.BlockSpec((BA, n_ang), lambda i: (i, 0)),
            pl.BlockSpec((BA, n_ang), lambda i: (i, 0)),
        ],
        out_shape=[
            jax.ShapeDtypeStruct((N, n_ang), jnp.float32),
            jax.ShapeDtypeStruct((N, n_ang), jnp.float32),
        ],
    )(pos, dsel, pxj, pyj, pzj, sj)

    return jnp.concatenate([radu, radw, angu, angw], axis=1)
